# baseline (speedup 1.0000x reference)
import functools

import jax
import jax.numpy as jnp
from jax import lax
from jax.experimental import pallas as pl
from jax.experimental.pallas import tpu as pltpu
from jax.experimental.pallas import tpu_sc as plsc

B = 16384
L = 20
EMB = 32
NC = 2
NS = 16
NW = NC * NS
BPW = B // NW
C = 128
NCH = BPW // C


def _sc_gather_pool(tid2, tok2, item_table, text_table):
    mesh = plsc.VectorSubcoreMesh(core_axis_name="c", subcore_axis_name="s")

    @functools.partial(
        pl.kernel,
        out_type=(
            jax.ShapeDtypeStruct((B, EMB), jnp.float32),
            jax.ShapeDtypeStruct((B, EMB), jnp.float32),
        ),
        mesh=mesh,
        compiler_params=pltpu.CompilerParams(use_tc_tiling_on_sc=False),
        scratch_types=[
            pltpu.VMEM((NCH, C), jnp.int32),
            pltpu.VMEM((BPW * L // 128, 128), jnp.int32),
            pltpu.VMEM((BPW, EMB), jnp.float32),
            pltpu.VMEM((C * L, EMB), jnp.float32),
            pltpu.VMEM((BPW, EMB), jnp.float32),
            pltpu.SemaphoreType.DMA,
            pltpu.SemaphoreType.DMA,
        ],
    )
    def k(tid_hbm, tok_hbm, item_hbm, text_hbm, ido_hbm, summ_hbm,
          tid_v, tok_v, item_v, gath_v, acc_v, sem_i, sem_g):
        wid = lax.axis_index("s") * NC + lax.axis_index("c")
        pltpu.sync_copy(tid_hbm.at[pl.ds(wid * NCH, NCH)], tid_v)
        pltpu.sync_copy(tok_hbm.at[pl.ds(wid * (BPW * L // 128),
                                         BPW * L // 128)], tok_v)
        item_cps = [
            pltpu.async_copy(item_hbm.at[tid_v.at[c]],
                             item_v.at[pl.ds(c * C, C)], sem_i)
            for c in range(NCH)
        ]
        for c in range(NCH):
            cps = [
                pltpu.async_copy(text_hbm.at[tok_v.at[c * L + a]],
                                 gath_v.at[pl.ds(a * C, C)], sem_g)
                for a in range(L)
            ]
            for cp in cps:
                cp.wait()

            def red(i, _, c=c):
                for h in range(2):
                    sl = pl.ds(h * 16, 16)
                    s = gath_v[i * L, sl]
                    for j in range(1, L):
                        s = s + gath_v[i * L + j, sl]
                    acc_v[c * C + i, sl] = s
                return 0

            lax.fori_loop(0, C, red, 0)
        for cp in item_cps:
            cp.wait()
        pltpu.sync_copy(item_v, ido_hbm.at[pl.ds(wid * BPW, BPW)])
        pltpu.sync_copy(acc_v, summ_hbm.at[pl.ds(wid * BPW, BPW)])

    return k(tid2, tok2, item_table, text_table)


def _tc_finalize(ido, summ, tok, t0):
    R = 2048

    def body(id_ref, sm_ref, tok_ref, t0_ref, o_ref):
        cnt = jnp.sum((tok_ref[...] != 0).astype(jnp.float32), axis=1,
                      keepdims=True)
        text = (sm_ref[...] - (L - cnt) * t0_ref[...]) / jnp.maximum(cnt, 1.0)
        o_ref[...] = jnp.concatenate([id_ref[...], text], axis=1)

    return pl.pallas_call(
        body,
        out_shape=jax.ShapeDtypeStruct((B, 2 * EMB), jnp.float32),
        grid=(B // R,),
        in_specs=[
            pl.BlockSpec((R, EMB), lambda i: (i, 0)),
            pl.BlockSpec((R, EMB), lambda i: (i, 0)),
            pl.BlockSpec((R, L), lambda i: (i, 0)),
            pl.BlockSpec((1, EMB), lambda i: (0, 0)),
        ],
        out_specs=pl.BlockSpec((R, 2 * EMB), lambda i: (i, 0)),
    )(ido, summ, tok, t0)


def kernel(title_ids, title_token_ids, item_table, text_table):
    tid2 = title_ids.reshape(NW * NCH, C)
    tok2 = title_token_ids.reshape(B * L // 128, 128)
    ido, summ = _sc_gather_pool(tid2, tok2, item_table, text_table)
    t0 = text_table[0:1, :]
    return _tc_finalize(ido, summ, title_token_ids, t0)


# speedup vs baseline: 1.2075x; 1.2075x over previous
import functools

import jax
import jax.numpy as jnp
from jax import lax
from jax.experimental import pallas as pl
from jax.experimental.pallas import tpu as pltpu
from jax.experimental.pallas import tpu_sc as plsc

B = 16384
L = 20
EMB = 32
NC = 2
NS = 16
NW = NC * NS
BPW = B // NW
C = 128
NCH = BPW // C
CT = 64
NCT = BPW // CT


def _sc_text_pool(tok, text_table):
    mesh = plsc.VectorSubcoreMesh(core_axis_name="c", subcore_axis_name="s")

    @functools.partial(
        pl.kernel,
        out_type=jax.ShapeDtypeStruct((B, EMB), jnp.float32),
        mesh=mesh,
        compiler_params=pltpu.CompilerParams(use_tc_tiling_on_sc=False),
        scratch_types=[
            pltpu.VMEM((BPW, L), jnp.int32),
            pltpu.VMEM((BPW * L + 16,), jnp.int32),
            pltpu.VMEM((CT * L, EMB), jnp.float32),
            pltpu.VMEM((BPW, EMB), jnp.float32),
            pltpu.SemaphoreType.DMA,
        ],
    )
    def k(tok_hbm, text_hbm, summ_hbm, tok_v, tok_f, gath_v, acc_v, sem_g):
        wid = lax.axis_index("s") * NC + lax.axis_index("c")
        pltpu.sync_copy(tok_hbm.at[pl.ds(wid * BPW, BPW)], tok_v)

        @plsc.parallel_loop(0, BPW, 1, unroll=4)
        def repack(i):
            tok_f[pl.ds(i * L, 16)] = tok_v[i, pl.ds(0, 16)]
            tok_f[pl.ds(i * L + (L - 16), 16)] = tok_v[i, pl.ds(L - 16, 16)]

        NSL = CT * L // 128
        for c in range(NCT):
            cps = [
                pltpu.async_copy(
                    text_hbm.at[tok_f.at[pl.ds(c * CT * L + a * 128, 128)]],
                    gath_v.at[pl.ds(a * 128, 128)], sem_g)
                for a in range(NSL)
            ]
            for cp in cps:
                cp.wait()

            @plsc.parallel_loop(0, CT, 1, unroll=2)
            def red(i, c=c):
                row = i * L
                for h in range(2):
                    sl = pl.ds(h * 16, 16)
                    t = [gath_v[row + j, sl] for j in range(L)]
                    while len(t) > 1:
                        t = [a + b for a, b in zip(t[::2], t[1::2])] + (
                            [t[-1]] if len(t) % 2 else [])
                    acc_v[c * CT + i, sl] = t[0]

        pltpu.sync_copy(acc_v, summ_hbm.at[pl.ds(wid * BPW, BPW)])

    return k(tok, text_table)


def _sc_item_gather(tid, item_table):
    mesh = plsc.VectorSubcoreMesh(core_axis_name="c", subcore_axis_name="s")

    @functools.partial(
        pl.kernel,
        out_type=jax.ShapeDtypeStruct((B, EMB), jnp.float32),
        mesh=mesh,
        compiler_params=pltpu.CompilerParams(use_tc_tiling_on_sc=False),
        scratch_types=[
            pltpu.VMEM((BPW,), jnp.int32),
            pltpu.VMEM((BPW, EMB), jnp.float32),
            pltpu.SemaphoreType.DMA,
        ],
    )
    def k(tid_hbm, item_hbm, ido_hbm, tid_v, item_v, sem_i):
        wid = lax.axis_index("s") * NC + lax.axis_index("c")
        pltpu.sync_copy(tid_hbm.at[pl.ds(wid * BPW, BPW)], tid_v)
        cps = [
            pltpu.async_copy(item_hbm.at[tid_v.at[pl.ds(c * C, C)]],
                             item_v.at[pl.ds(c * C, C)], sem_i)
            for c in range(NCH)
        ]
        for cp in cps:
            cp.wait()
        pltpu.sync_copy(item_v, ido_hbm.at[pl.ds(wid * BPW, BPW)])

    return k(tid, item_table)


def _tc_finalize(ido, summ, tok, t0):
    R = 2048

    def body(id_ref, sm_ref, tok_ref, t0_ref, o_ref):
        cnt = jnp.sum((tok_ref[...] != 0).astype(jnp.float32), axis=1,
                      keepdims=True)
        text = (sm_ref[...] - (L - cnt) * t0_ref[...]) / jnp.maximum(cnt, 1.0)
        o_ref[...] = jnp.concatenate([id_ref[...], text], axis=1)

    return pl.pallas_call(
        body,
        out_shape=jax.ShapeDtypeStruct((B, 2 * EMB), jnp.float32),
        grid=(B // R,),
        in_specs=[
            pl.BlockSpec((R, EMB), lambda i: (i, 0)),
            pl.BlockSpec((R, EMB), lambda i: (i, 0)),
            pl.BlockSpec((R, L), lambda i: (i, 0)),
            pl.BlockSpec((1, EMB), lambda i: (0, 0)),
        ],
        out_specs=pl.BlockSpec((R, 2 * EMB), lambda i: (i, 0)),
    )(ido, summ, tok, t0)


def kernel(title_ids, title_token_ids, item_table, text_table):
    summ = _sc_text_pool(title_token_ids, text_table)
    ido = _sc_item_gather(title_ids, item_table)
    t0 = text_table[0:1, :]
    return _tc_finalize(ido, summ, title_token_ids, t0)


# speedup vs baseline: 1.2695x; 1.0513x over previous
import functools

import jax
import jax.numpy as jnp
from jax import lax
from jax.experimental import pallas as pl
from jax.experimental.pallas import tpu as pltpu
from jax.experimental.pallas import tpu_sc as plsc

B = 16384
L = 20
EMB = 32
NC = 2
NS = 16
NW = NC * NS
BPW = B // NW
C = 128
NCH = BPW // C
CT = 64
NCT = BPW // CT


def _sc_text_pool(tok, text_table):
    mesh = plsc.VectorSubcoreMesh(core_axis_name="c", subcore_axis_name="s")

    @functools.partial(
        pl.kernel,
        out_type=jax.ShapeDtypeStruct((B, EMB), jnp.float32),
        mesh=mesh,
        compiler_params=pltpu.CompilerParams(use_tc_tiling_on_sc=False,
                                             needs_layout_passes=False),
        scratch_types=[
            pltpu.VMEM((BPW, L), jnp.int32),
            pltpu.VMEM((BPW * L + 16,), jnp.int32),
            pltpu.VMEM((BPW, 16), jnp.float32),
            pltpu.VMEM((1, EMB), jnp.float32),
            pltpu.VMEM((CT * L, EMB), jnp.float32),
            pltpu.VMEM((BPW, EMB), jnp.float32),
            pltpu.SemaphoreType.DMA,
        ],
    )
    def k(tok_hbm, text_hbm, summ_hbm, tok_v, tok_f, cnt_v, t0_v, gath_v,
          acc_v, sem_g):
        wid = lax.axis_index("s") * NC + lax.axis_index("c")
        pltpu.sync_copy(tok_hbm.at[pl.ds(wid * BPW, BPW)], tok_v)
        pltpu.sync_copy(text_hbm.at[pl.ds(0, 1)], t0_v)
        tail_sel = lax.iota(jnp.int32, 16) >= 2 * 16 - L

        @plsc.parallel_loop(0, BPW, 1, unroll=4)
        def repack(i):
            v0 = tok_v[i, pl.ds(0, 16)]
            v1 = tok_v[i, pl.ds(L - 16, 16)]
            tok_f[pl.ds(i * L, 16)] = v0
            tok_f[pl.ds(i * L + (L - 16), 16)] = v1
            nz0 = plsc.all_reduce_population_count(v0 != 0)
            nz1 = plsc.all_reduce_population_count(
                jnp.logical_and(v1 != 0, tail_sel))
            cnt_v[i, pl.ds(0, 16)] = (nz0 + nz1).astype(jnp.float32)

        t00 = t0_v[0, pl.ds(0, 16)]
        t01 = t0_v[0, pl.ds(16, 16)]
        NSL = CT * L // 128
        for c in range(NCT):
            cps = [
                pltpu.async_copy(
                    text_hbm.at[tok_f.at[pl.ds(c * CT * L + a * 128, 128)]],
                    gath_v.at[pl.ds(a * 128, 128)], sem_g)
                for a in range(NSL)
            ]
            for cp in cps:
                cp.wait()

            @plsc.parallel_loop(0, CT, 1, unroll=2)
            def red(i, c=c):
                row = i * L
                cnt = cnt_v[c * CT + i, pl.ds(0, 16)]
                n0 = float(L) - cnt
                inv = 1.0 / jnp.maximum(cnt, 1.0)
                for h, t0h in ((0, t00), (1, t01)):
                    sl = pl.ds(h * 16, 16)
                    t = [gath_v[row + j, sl] for j in range(L)]
                    while len(t) > 1:
                        t = [a + b for a, b in zip(t[::2], t[1::2])] + (
                            [t[-1]] if len(t) % 2 else [])
                    acc_v[c * CT + i, sl] = (t[0] - n0 * t0h) * inv

        pltpu.sync_copy(acc_v, summ_hbm.at[pl.ds(wid * BPW, BPW)])

    return k(tok, text_table)


def _sc_item_gather(tid, item_table):
    mesh = plsc.VectorSubcoreMesh(core_axis_name="c", subcore_axis_name="s")

    @functools.partial(
        pl.kernel,
        out_type=jax.ShapeDtypeStruct((B, EMB), jnp.float32),
        mesh=mesh,
        compiler_params=pltpu.CompilerParams(use_tc_tiling_on_sc=False),
        scratch_types=[
            pltpu.VMEM((BPW,), jnp.int32),
            pltpu.VMEM((BPW, EMB), jnp.float32),
            pltpu.SemaphoreType.DMA,
        ],
    )
    def k(tid_hbm, item_hbm, ido_hbm, tid_v, item_v, sem_i):
        wid = lax.axis_index("s") * NC + lax.axis_index("c")
        pltpu.sync_copy(tid_hbm.at[pl.ds(wid * BPW, BPW)], tid_v)
        cps = [
            pltpu.async_copy(item_hbm.at[tid_v.at[pl.ds(c * C, C)]],
                             item_v.at[pl.ds(c * C, C)], sem_i)
            for c in range(NCH)
        ]
        for cp in cps:
            cp.wait()
        pltpu.sync_copy(item_v, ido_hbm.at[pl.ds(wid * BPW, BPW)])

    return k(tid, item_table)


def _tc_finalize(ido, summ, tok, t0):
    R = 2048

    def body(id_ref, sm_ref, tok_ref, t0_ref, o_ref):
        cnt = jnp.sum((tok_ref[...] != 0).astype(jnp.float32), axis=1,
                      keepdims=True)
        text = (sm_ref[...] - (L - cnt) * t0_ref[...]) / jnp.maximum(cnt, 1.0)
        o_ref[...] = jnp.concatenate([id_ref[...], text], axis=1)

    return pl.pallas_call(
        body,
        out_shape=jax.ShapeDtypeStruct((B, 2 * EMB), jnp.float32),
        grid=(B // R,),
        in_specs=[
            pl.BlockSpec((R, EMB), lambda i: (i, 0)),
            pl.BlockSpec((R, EMB), lambda i: (i, 0)),
            pl.BlockSpec((R, L), lambda i: (i, 0)),
            pl.BlockSpec((1, EMB), lambda i: (0, 0)),
        ],
        out_specs=pl.BlockSpec((R, 2 * EMB), lambda i: (i, 0)),
    )(ido, summ, tok, t0)


def kernel(title_ids, title_token_ids, item_table, text_table):
    text_emb = _sc_text_pool(title_token_ids, text_table)
    ido = _sc_item_gather(title_ids, item_table)
    return jnp.concatenate([ido, text_emb], axis=1)


# speedup vs baseline: 1.3108x; 1.0326x over previous
import functools

import jax
import jax.numpy as jnp
from jax import lax
from jax.experimental import pallas as pl
from jax.experimental.pallas import tpu as pltpu
from jax.experimental.pallas import tpu_sc as plsc

B = 16384
L = 20
EMB = 32
NC = 2
NS = 16
NW = NC * NS
BPW = B // NW
C = 128
NCH = BPW // C
CT = 64
NCT = BPW // CT


def _sc_text_pool(tok, text_table):
    mesh = plsc.VectorSubcoreMesh(core_axis_name="c", subcore_axis_name="s")

    @functools.partial(
        pl.kernel,
        out_type=jax.ShapeDtypeStruct((B, EMB), jnp.float32),
        mesh=mesh,
        compiler_params=pltpu.CompilerParams(use_tc_tiling_on_sc=False,
                                             needs_layout_passes=False),
        scratch_types=[
            pltpu.VMEM((BPW, L), jnp.int32),
            pltpu.VMEM((BPW * L + 16,), jnp.int32),
            pltpu.VMEM((BPW, 16), jnp.float32),
            pltpu.VMEM((1, EMB), jnp.float32),
            pltpu.VMEM((CT * L, EMB), jnp.float32),
            pltpu.VMEM((CT * L, EMB), jnp.float32),
            pltpu.VMEM((BPW, EMB), jnp.float32),
            pltpu.SemaphoreType.DMA,
            pltpu.SemaphoreType.DMA,
        ],
    )
    def k(tok_hbm, text_hbm, summ_hbm, tok_v, tok_f, cnt_v, t0_v, gath_a,
          gath_b, acc_v, sem_a, sem_b):
        wid = lax.axis_index("s") * NC + lax.axis_index("c")
        pltpu.sync_copy(tok_hbm.at[pl.ds(wid * BPW, BPW)], tok_v)
        pltpu.sync_copy(text_hbm.at[pl.ds(0, 1)], t0_v)
        tail_sel = lax.iota(jnp.int32, 16) >= 2 * 16 - L

        @plsc.parallel_loop(0, BPW, 1, unroll=4)
        def repack(i):
            v0 = tok_v[i, pl.ds(0, 16)]
            v1 = tok_v[i, pl.ds(L - 16, 16)]
            tok_f[pl.ds(i * L, 16)] = v0
            tok_f[pl.ds(i * L + (L - 16), 16)] = v1
            nz0 = plsc.all_reduce_population_count(v0 != 0)
            nz1 = plsc.all_reduce_population_count(
                jnp.logical_and(v1 != 0, tail_sel))
            cnt_v[i, pl.ds(0, 16)] = (nz0 + nz1).astype(jnp.float32)

        t00 = t0_v[0, pl.ds(0, 16)]
        t01 = t0_v[0, pl.ds(16, 16)]
        NSL = CT * L // 128
        bufs = (gath_a, gath_b)
        sems = (sem_a, sem_b)

        def fire(c):
            return [
                pltpu.async_copy(
                    text_hbm.at[tok_f.at[pl.ds(c * CT * L + a * 128, 128)]],
                    bufs[c % 2].at[pl.ds(a * 128, 128)], sems[c % 2])
                for a in range(NSL)
            ]

        pend = fire(0)
        for c in range(NCT):
            nxt = fire(c + 1) if c + 1 < NCT else []
            for cp in pend:
                cp.wait()
            pend = nxt
            gath_v = bufs[c % 2]

            @plsc.parallel_loop(0, CT, 1, unroll=2)
            def red(i, c=c, gath_v=gath_v):
                row = i * L
                cnt = cnt_v[c * CT + i, pl.ds(0, 16)]
                n0 = float(L) - cnt
                inv = 1.0 / jnp.maximum(cnt, 1.0)
                for h, t0h in ((0, t00), (1, t01)):
                    sl = pl.ds(h * 16, 16)
                    t = [gath_v[row + j, sl] for j in range(L)]
                    while len(t) > 1:
                        t = [a + b for a, b in zip(t[::2], t[1::2])] + (
                            [t[-1]] if len(t) % 2 else [])
                    acc_v[c * CT + i, sl] = (t[0] - n0 * t0h) * inv

        pltpu.sync_copy(acc_v, summ_hbm.at[pl.ds(wid * BPW, BPW)])

    return k(tok, text_table)


def _sc_item_gather(tid, item_table):
    mesh = plsc.VectorSubcoreMesh(core_axis_name="c", subcore_axis_name="s")

    @functools.partial(
        pl.kernel,
        out_type=jax.ShapeDtypeStruct((B, EMB), jnp.float32),
        mesh=mesh,
        compiler_params=pltpu.CompilerParams(use_tc_tiling_on_sc=False),
        scratch_types=[
            pltpu.VMEM((BPW,), jnp.int32),
            pltpu.VMEM((BPW, EMB), jnp.float32),
            pltpu.SemaphoreType.DMA,
        ],
    )
    def k(tid_hbm, item_hbm, ido_hbm, tid_v, item_v, sem_i):
        wid = lax.axis_index("s") * NC + lax.axis_index("c")
        pltpu.sync_copy(tid_hbm.at[pl.ds(wid * BPW, BPW)], tid_v)
        cps = [
            pltpu.async_copy(item_hbm.at[tid_v.at[pl.ds(c * C, C)]],
                             item_v.at[pl.ds(c * C, C)], sem_i)
            for c in range(NCH)
        ]
        for cp in cps:
            cp.wait()
        pltpu.sync_copy(item_v, ido_hbm.at[pl.ds(wid * BPW, BPW)])

    return k(tid, item_table)


def _tc_finalize(ido, summ, tok, t0):
    R = 2048

    def body(id_ref, sm_ref, tok_ref, t0_ref, o_ref):
        cnt = jnp.sum((tok_ref[...] != 0).astype(jnp.float32), axis=1,
                      keepdims=True)
        text = (sm_ref[...] - (L - cnt) * t0_ref[...]) / jnp.maximum(cnt, 1.0)
        o_ref[...] = jnp.concatenate([id_ref[...], text], axis=1)

    return pl.pallas_call(
        body,
        out_shape=jax.ShapeDtypeStruct((B, 2 * EMB), jnp.float32),
        grid=(B // R,),
        in_specs=[
            pl.BlockSpec((R, EMB), lambda i: (i, 0)),
            pl.BlockSpec((R, EMB), lambda i: (i, 0)),
            pl.BlockSpec((R, L), lambda i: (i, 0)),
            pl.BlockSpec((1, EMB), lambda i: (0, 0)),
        ],
        out_specs=pl.BlockSpec((R, 2 * EMB), lambda i: (i, 0)),
    )(ido, summ, tok, t0)


def kernel(title_ids, title_token_ids, item_table, text_table):
    text_emb = _sc_text_pool(title_token_ids, text_table)
    ido = _sc_item_gather(title_ids, item_table)
    return jnp.concatenate([ido, text_emb], axis=1)


# speedup vs baseline: 1.3133x; 1.0019x over previous
import functools

import jax
import jax.numpy as jnp
from jax import lax
from jax.experimental import pallas as pl
from jax.experimental.pallas import tpu as pltpu
from jax.experimental.pallas import tpu_sc as plsc

B = 16384
L = 20
EMB = 32
NC = 2
NS = 16
NW = NC * NS
BPW = B // NW
C = 128
NCH = BPW // C
CT = 64
NCT = BPW // CT


def _sc_text_pool(tok, text_table):
    mesh = plsc.VectorSubcoreMesh(core_axis_name="c", subcore_axis_name="s")

    @functools.partial(
        pl.kernel,
        out_type=jax.ShapeDtypeStruct((B, EMB), jnp.float32),
        mesh=mesh,
        compiler_params=pltpu.CompilerParams(use_tc_tiling_on_sc=False,
                                             needs_layout_passes=False),
        scratch_types=[
            pltpu.VMEM((BPW, L), jnp.int32),
            pltpu.VMEM((BPW * L + 16,), jnp.int32),
            pltpu.VMEM((BPW, 16), jnp.float32),
            pltpu.VMEM((1, EMB), jnp.float32),
            pltpu.VMEM((CT * L, EMB), jnp.float32),
            pltpu.VMEM((CT * L, EMB), jnp.float32),
            pltpu.VMEM((BPW, EMB), jnp.float32),
            pltpu.SemaphoreType.DMA,
            pltpu.SemaphoreType.DMA,
        ],
    )
    def k(tok_hbm, text_hbm, summ_hbm, tok_v, tok_f, cnt_v, t0_v, gath_a,
          gath_b, acc_v, sem_a, sem_b):
        wid = lax.axis_index("s") * NC + lax.axis_index("c")
        pltpu.sync_copy(tok_hbm.at[pl.ds(wid * BPW, BPW)], tok_v)
        pltpu.sync_copy(text_hbm.at[pl.ds(0, 1)], t0_v)
        tail_sel = lax.iota(jnp.int32, 16) >= 2 * 16 - L

        @plsc.parallel_loop(0, BPW, 1, unroll=4)
        def repack(i):
            v0 = tok_v[i, pl.ds(0, 16)]
            v1 = tok_v[i, pl.ds(L - 16, 16)]
            tok_f[pl.ds(i * L, 16)] = v0
            tok_f[pl.ds(i * L + (L - 16), 16)] = v1
            nz0 = plsc.all_reduce_population_count(v0 != 0)
            nz1 = plsc.all_reduce_population_count(
                jnp.logical_and(v1 != 0, tail_sel))
            cnt_v[i, pl.ds(0, 16)] = (nz0 + nz1).astype(jnp.float32)

        t00 = t0_v[0, pl.ds(0, 16)]
        t01 = t0_v[0, pl.ds(16, 16)]
        NSL = CT * L // 128
        bufs = (gath_a, gath_b)
        sems = (sem_a, sem_b)

        def fire(c):
            return [
                pltpu.async_copy(
                    text_hbm.at[tok_f.at[pl.ds(c * CT * L + a * 128, 128)]],
                    bufs[c % 2].at[pl.ds(a * 128, 128)], sems[c % 2])
                for a in range(NSL)
            ]

        pend = fire(0)
        for c in range(NCT):
            nxt = fire(c + 1) if c + 1 < NCT else []
            for cp in pend:
                cp.wait()
            pend = nxt
            gath_v = bufs[c % 2]

            @plsc.parallel_loop(0, CT, 1, unroll=4)
            def red(i, c=c, gath_v=gath_v):
                row = i * L
                cnt = cnt_v[c * CT + i, pl.ds(0, 16)]
                n0 = float(L) - cnt
                inv = 1.0 / jnp.maximum(cnt, 1.0)
                for h, t0h in ((0, t00), (1, t01)):
                    sl = pl.ds(h * 16, 16)
                    t = [gath_v[row + j, sl] for j in range(L)]
                    while len(t) > 1:
                        t = [a + b for a, b in zip(t[::2], t[1::2])] + (
                            [t[-1]] if len(t) % 2 else [])
                    acc_v[c * CT + i, sl] = (t[0] - n0 * t0h) * inv

        pltpu.sync_copy(acc_v, summ_hbm.at[pl.ds(wid * BPW, BPW)])

    return k(tok, text_table)


def _sc_item_gather(tid, item_table):
    mesh = plsc.VectorSubcoreMesh(core_axis_name="c", subcore_axis_name="s")

    @functools.partial(
        pl.kernel,
        out_type=jax.ShapeDtypeStruct((B, EMB), jnp.float32),
        mesh=mesh,
        compiler_params=pltpu.CompilerParams(use_tc_tiling_on_sc=False),
        scratch_types=[
            pltpu.VMEM((BPW,), jnp.int32),
            pltpu.VMEM((BPW, EMB), jnp.float32),
            pltpu.SemaphoreType.DMA,
        ],
    )
    def k(tid_hbm, item_hbm, ido_hbm, tid_v, item_v, sem_i):
        wid = lax.axis_index("s") * NC + lax.axis_index("c")
        pltpu.sync_copy(tid_hbm.at[pl.ds(wid * BPW, BPW)], tid_v)
        cps = [
            pltpu.async_copy(item_hbm.at[tid_v.at[pl.ds(c * C, C)]],
                             item_v.at[pl.ds(c * C, C)], sem_i)
            for c in range(NCH)
        ]
        for cp in cps:
            cp.wait()
        pltpu.sync_copy(item_v, ido_hbm.at[pl.ds(wid * BPW, BPW)])

    return k(tid, item_table)


def _tc_finalize(ido, summ, tok, t0):
    R = 2048

    def body(id_ref, sm_ref, tok_ref, t0_ref, o_ref):
        cnt = jnp.sum((tok_ref[...] != 0).astype(jnp.float32), axis=1,
                      keepdims=True)
        text = (sm_ref[...] - (L - cnt) * t0_ref[...]) / jnp.maximum(cnt, 1.0)
        o_ref[...] = jnp.concatenate([id_ref[...], text], axis=1)

    return pl.pallas_call(
        body,
        out_shape=jax.ShapeDtypeStruct((B, 2 * EMB), jnp.float32),
        grid=(B // R,),
        in_specs=[
            pl.BlockSpec((R, EMB), lambda i: (i, 0)),
            pl.BlockSpec((R, EMB), lambda i: (i, 0)),
            pl.BlockSpec((R, L), lambda i: (i, 0)),
            pl.BlockSpec((1, EMB), lambda i: (0, 0)),
        ],
        out_specs=pl.BlockSpec((R, 2 * EMB), lambda i: (i, 0)),
    )(ido, summ, tok, t0)


def kernel(title_ids, title_token_ids, item_table, text_table):
    text_emb = _sc_text_pool(title_token_ids, text_table)
    ido = _sc_item_gather(title_ids, item_table)
    return jnp.concatenate([ido, text_emb], axis=1)


# speedup vs baseline: 1.3174x; 1.0031x over previous
import functools

import jax
import jax.numpy as jnp
from jax import lax
from jax.experimental import pallas as pl
from jax.experimental.pallas import tpu as pltpu
from jax.experimental.pallas import tpu_sc as plsc

B = 16384
L = 20
EMB = 32
NC = 2
NS = 16
NW = NC * NS
BPW = B // NW
C = 128
NCH = BPW // C
CT = 64
NCT = BPW // CT


def _sc_text_pool(tok, text_table):
    mesh = plsc.VectorSubcoreMesh(core_axis_name="c", subcore_axis_name="s")

    @functools.partial(
        pl.kernel,
        out_type=jax.ShapeDtypeStruct((B, EMB), jnp.float32),
        mesh=mesh,
        compiler_params=pltpu.CompilerParams(use_tc_tiling_on_sc=False,
                                             needs_layout_passes=False),
        scratch_types=[
            pltpu.VMEM((BPW, L), jnp.int32),
            pltpu.VMEM((BPW * L + 16,), jnp.int32),
            pltpu.VMEM((BPW, 16), jnp.float32),
            pltpu.VMEM((1, EMB), jnp.float32),
            pltpu.VMEM((CT * L, EMB), jnp.float32),
            pltpu.VMEM((CT * L, EMB), jnp.float32),
            pltpu.VMEM((BPW, EMB), jnp.float32),
            pltpu.SemaphoreType.DMA,
            pltpu.SemaphoreType.DMA,
        ],
    )
    def k(tok_hbm, text_hbm, summ_hbm, tok_v, tok_f, cnt_v, t0_v, gath_a,
          gath_b, acc_v, sem_a, sem_b):
        wid = lax.axis_index("s") * NC + lax.axis_index("c")
        pltpu.sync_copy(tok_hbm.at[pl.ds(wid * BPW, BPW)], tok_v)
        pltpu.sync_copy(text_hbm.at[pl.ds(0, 1)], t0_v)
        tail_sel = lax.iota(jnp.int32, 16) >= 2 * 16 - L

        @plsc.parallel_loop(0, BPW, 1, unroll=4)
        def repack(i):
            v0 = tok_v[i, pl.ds(0, 16)]
            v1 = tok_v[i, pl.ds(L - 16, 16)]
            tok_f[pl.ds(i * L, 16)] = v0
            tok_f[pl.ds(i * L + (L - 16), 16)] = v1
            nz0 = plsc.all_reduce_population_count(v0 != 0)
            nz1 = plsc.all_reduce_population_count(
                jnp.logical_and(v1 != 0, tail_sel))
            cnt_v[i, pl.ds(0, 16)] = (nz0 + nz1).astype(jnp.float32)

        t00 = t0_v[0, pl.ds(0, 16)]
        t01 = t0_v[0, pl.ds(16, 16)]
        NSL = CT * L // 128
        bufs = (gath_a, gath_b)
        sems = (sem_a, sem_b)

        def fire(c):
            return [
                pltpu.async_copy(
                    text_hbm.at[tok_f.at[pl.ds(c * CT * L + a * 128, 128)]],
                    bufs[c % 2].at[pl.ds(a * 128, 128)], sems[c % 2])
                for a in range(NSL)
            ]

        pend = fire(0)
        for c in range(NCT):
            nxt = fire(c + 1) if c + 1 < NCT else []
            for cp in pend:
                cp.wait()
            pend = nxt
            gath_v = bufs[c % 2]

            @plsc.parallel_loop(0, CT, 1, unroll=4)
            def red(i, c=c, gath_v=gath_v):
                row = i * L
                cnt = cnt_v[c * CT + i, pl.ds(0, 16)]
                n0 = float(L) - cnt
                inv = 1.0 / jnp.maximum(cnt, 1.0)
                for h, t0h in ((0, t00), (1, t01)):
                    sl = pl.ds(h * 16, 16)
                    t = [gath_v[row + j, sl] for j in range(L)]
                    while len(t) > 1:
                        t = [a + b for a, b in zip(t[::2], t[1::2])] + (
                            [t[-1]] if len(t) % 2 else [])
                    acc_v[c * CT + i, sl] = (t[0] - n0 * t0h) * inv

        pltpu.sync_copy(acc_v, summ_hbm.at[pl.ds(wid * BPW, BPW)])

    return k(tok, text_table)


def _sc_item_gather(tid, item_table):
    mesh = plsc.VectorSubcoreMesh(core_axis_name="c", subcore_axis_name="s")

    @functools.partial(
        pl.kernel,
        out_type=jax.ShapeDtypeStruct((B, EMB), jnp.float32),
        mesh=mesh,
        compiler_params=pltpu.CompilerParams(use_tc_tiling_on_sc=False),
        scratch_types=[
            pltpu.VMEM((BPW,), jnp.int32),
            pltpu.VMEM((BPW, EMB), jnp.float32),
            pltpu.SemaphoreType.DMA,
        ],
    )
    def k(tid_hbm, item_hbm, ido_hbm, tid_v, item_v, sem_i):
        wid = lax.axis_index("s") * NC + lax.axis_index("c")
        pltpu.sync_copy(tid_hbm.at[pl.ds(wid * BPW, BPW)], tid_v)
        cps = [
            pltpu.async_copy(item_hbm.at[tid_v.at[pl.ds(c * C, C)]],
                             item_v.at[pl.ds(c * C, C)], sem_i)
            for c in range(NCH)
        ]
        for cp in cps:
            cp.wait()
        pltpu.sync_copy(item_v, ido_hbm.at[pl.ds(wid * BPW, BPW)])

    return k(tid, item_table)


def kernel(title_ids, title_token_ids, item_table, text_table):
    text_emb = _sc_text_pool(title_token_ids, text_table)
    ido = _sc_item_gather(title_ids, item_table)
    return jnp.concatenate([ido, text_emb], axis=1)
